# baseline (speedup 1.0000x reference)
import jax
import jax.numpy as jnp
from jax import lax
from jax.experimental import pallas as pl
from jax.experimental.pallas import tpu as pltpu

B, N, K, H = 4, 1024, 2, 4
D_EMB, D_FEAT, D_ENC = 128, 64, 256
DH = D_ENC // H
GRAPH_LAYERS, FLOW_ITERS = 2, 10
DUAL_STEP, DUAL_MOM, DUAL_ITERS = 0.01, 0.9, 10
BIG = 1e7
BISECT_ITERS = 16
_EXPC = (0.9999998618921861, 0.999999927064532, 0.5000049645882827,
         0.16666773498127707, 0.04163942807486006, 0.008329176014426213,
         0.0014358722593905257, 0.00020432579377860398)
PREC = lax.Precision.DEFAULT


def _fwd_kernel(x_ref, dem_ref, adj_ref, nbr_ref,
                encW1, encb1, encW2, encb2,
                Wv, a_src, a_dst, Wcomb, bcomb,
                Wz, Uz, bz, Wr, Ur, br, Wh, Uh, bh,
                decW1, decb1, decW2, decb2,
                duW1, dub1, duW2, dub2,
                out_ref):
    x = x_ref[0]
    h0 = jnp.tanh(jnp.dot(x, encW1[...], precision=PREC) + encb1[...])
    enc = jnp.tanh(jnp.dot(h0, encW2[...], precision=PREC) + encb2[...])

    for _ in range(GRAPH_LAYERS):
        acc = jnp.zeros((N, D_ENC), jnp.float32)
        for k in range(K):
            mask = nbr_ref[0, k]
            for hh in range(H):
                i = k * H + hh
                v = jnp.tanh(jnp.dot(enc, Wv[i], precision=PREC))
                a_s = a_src[i:i + 1, :]
                a_d = a_dst[i:i + 1, :]
                s_src = lax.dot_general(v, a_s, (((1,), (1,)), ((), ())), precision=PREC)
                s_dst = lax.dot_general(a_d, v, (((1,), (1,)), ((), ())), precision=PREC)
                t = jnp.tanh(s_src + s_dst)
                ex = _EXPC[7]
                for c in (_EXPC[6], _EXPC[5], _EXPC[4], _EXPC[3],
                          _EXPC[2], _EXPC[1], _EXPC[0]):
                    ex = ex * t + c
                ex = ex * mask
                v1 = jnp.concatenate([v, jnp.ones((N, DH), jnp.float32)], axis=1)
                hv1 = jnp.dot(ex, v1, precision=PREC)
                hv = hv1[:, :DH] / (hv1[:, DH:DH + 1] + 1e-9)
                acc = acc + jnp.dot(hv, Wcomb[i * DH:(i + 1) * DH, :], precision=PREC)
        nxt = jnp.tanh(acc + bcomb[...])
        z = jax.nn.sigmoid(jnp.dot(nxt, Wz[...], precision=PREC) + jnp.dot(enc, Uz[...], precision=PREC) + bz[...])
        r = jax.nn.sigmoid(jnp.dot(nxt, Wr[...], precision=PREC) + jnp.dot(enc, Ur[...], precision=PREC) + br[...])
        c = jnp.tanh(jnp.dot(nxt, Wh[...], precision=PREC) + jnp.dot(r * enc, Uh[...], precision=PREC) + bh[...])
        enc = z * enc + (1.0 - z) * c

    adjm = adj_ref[0]
    dh = jnp.tanh(jnp.dot(enc, decW1[...], precision=PREC) + decb1[...])
    pred_row = lax.dot_general(decW2[...], dh, (((0,), (1,)), ((), ())), precision=PREC) + decb2[...]
    w = adjm * pred_row - BIG * (1.0 - adjm)

    zmax = jnp.max(w, axis=1, keepdims=True)
    lo = zmax - 1.0
    hi = zmax
    for _ in range(BISECT_ITERS):
        mid = 0.5 * (lo + hi)
        fsum = jnp.sum(jnp.maximum(w - mid, 0.0), axis=1, keepdims=True)
        gt = fsum > 1.0
        lo = jnp.where(gt, mid, lo)
        hi = jnp.where(gt, hi, mid)
    sup = (w > lo).astype(jnp.float32)
    kk = jnp.sum(sup, axis=1, keepdims=True)
    ss = jnp.sum(w * sup, axis=1, keepdims=True)
    tau = (ss - 1.0) / kk
    fw = jnp.maximum(w - tau, 0.0) * adjm

    dem_row = dem_ref[0]
    r_row = jnp.maximum(dem_row, 0.0)
    for _ in range(FLOW_ITERS - 1):
        r_col = jnp.transpose(r_row)
        inflow = jnp.sum(fw * r_col, axis=0, keepdims=True)
        r_row = jnp.maximum(inflow + dem_row, 0.0)
    r_col = jnp.transpose(r_row)
    flow_cost = jnp.sum((fw * fw) * (r_col * r_col))

    du = jnp.tanh(jnp.dot(enc, duW1[...], precision=PREC) + dub1[...])
    dv_col = jnp.dot(du, duW2[...], precision=PREC) + dub2[...]
    dv_row = lax.dot_general(duW2[...], du, (((0,), (1,)), ((), ())), precision=PREC) + dub2[...]
    dd = adjm * (dv_col - dv_row)
    dd01 = DUAL_STEP * dd
    fl = jnp.zeros((N, N), jnp.float32)
    vel = jnp.zeros((N, N), jnp.float32)
    for _ in range(DUAL_ITERS):
        vel = DUAL_MOM * vel + (dd01 - (2.0 * DUAL_STEP) * fl)
        fl = jnp.maximum(fl + vel, 0.0)
    dual_demand = jnp.sum(dv_row * dem_row)
    dual_cost = jnp.sum(fl * fl - dd * fl) - dual_demand
    out_ref[...] = jnp.broadcast_to(flow_cost - dual_cost, (1, 1, 128))


def kernel(node_features, node_embeddings, demands, adj, neighborhoods, params):
    p = params
    x = jnp.concatenate([node_embeddings, node_features], axis=-1)
    dem_row = jnp.transpose(demands, (0, 2, 1))
    Wv = p['Wv'].reshape(K * H, D_ENC, DH)
    a_src = p['a_src'].reshape(K * H, DH)
    a_dst = p['a_dst'].reshape(K * H, DH)

    def b2(b):
        return b.reshape(1, -1)

    args = [x, dem_row, adj, neighborhoods,
            p['enc_W1'], b2(p['enc_b1']), p['enc_W2'], b2(p['enc_b2']),
            Wv, a_src, a_dst, p['W_comb'], b2(p['b_comb']),
            p['Wz'], p['Uz'], b2(p['bz']), p['Wr'], p['Ur'], b2(p['br']),
            p['Wh'], p['Uh'], b2(p['bh']),
            p['dec_W1'], b2(p['dec_b1']), p['dec_W2'], b2(p['dec_b2']),
            p['dual_W1'], b2(p['dual_b1']), p['dual_W2'], b2(p['dual_b2'])]

    n_batch_args = 4
    in_specs = []
    for idx, a in enumerate(args):
        if idx < n_batch_args:
            blk = (1,) + a.shape[1:]
            in_specs.append(
                pl.BlockSpec(blk, lambda b, _nd=a.ndim: (b,) + (0,) * (_nd - 1)))
        else:
            in_specs.append(
                pl.BlockSpec(a.shape, lambda b, _nd=a.ndim: (0,) * _nd))

    out = pl.pallas_call(
        _fwd_kernel,
        grid=(B,),
        in_specs=in_specs,
        out_specs=pl.BlockSpec((1, 1, 128), lambda b: (b, 0, 0)),
        out_shape=jax.ShapeDtypeStruct((B, 1, 128), jnp.float32),
        compiler_params=pltpu.CompilerParams(dimension_semantics=("parallel",)),
    )(*args)
    return out[:, 0, 0]


# speedup vs baseline: 1.3227x; 1.3227x over previous
import jax
import jax.numpy as jnp
from jax import lax
from jax.experimental import pallas as pl
from jax.experimental.pallas import tpu as pltpu

B, N, K, H = 4, 1024, 2, 4
D_EMB, D_FEAT, D_ENC = 128, 64, 256
DH = D_ENC // H
GRAPH_LAYERS, FLOW_ITERS = 2, 10
DUAL_STEP, DUAL_MOM, DUAL_ITERS = 0.01, 0.9, 10
BIG = 1e7
BISECT_ITERS = 16
PREC = lax.Precision.DEFAULT


def _fwd_kernel(x_ref, dem_ref, adj_ref, nbr_ref,
                encW1, encb1, encW2, encb2,
                Wv, a_src, a_dst, Wcomb, bcomb,
                Wz, Uz, bz, Wr, Ur, br, Wh, Uh, bh,
                decW1, decb1, decW2, decb2,
                duW1, dub1, duW2, dub2,
                out_ref):
    x = x_ref[0]
    h0 = jnp.tanh(jnp.dot(x, encW1[...], precision=PREC) + encb1[...])
    enc = jnp.tanh(jnp.dot(h0, encW2[...], precision=PREC) + encb2[...])

    for _ in range(GRAPH_LAYERS):
        acc = jnp.zeros((N, D_ENC), jnp.float32)
        for k in range(K):
            mask = nbr_ref[0, k]
            for hh in range(H):
                i = k * H + hh
                v = jnp.tanh(jnp.dot(enc, Wv[i], precision=PREC))
                a_s = a_src[i:i + 1, :]
                a_d = a_dst[i:i + 1, :]
                s_src = lax.dot_general(v, a_s, (((1,), (1,)), ((), ())), precision=PREC)
                s_dst = lax.dot_general(a_d, v, (((1,), (1,)), ((), ())), precision=PREC)
                ex = jnp.exp(jnp.tanh(s_src + s_dst)) * mask
                v1 = jnp.concatenate([v, jnp.ones((N, DH), jnp.float32)], axis=1)
                hv1 = jnp.dot(ex, v1, precision=PREC)
                hv = hv1[:, :DH] / (hv1[:, DH:DH + 1] + 1e-9)
                acc = acc + jnp.dot(hv, Wcomb[i * DH:(i + 1) * DH, :], precision=PREC)
        nxt = jnp.tanh(acc + bcomb[...])
        z = jax.nn.sigmoid(jnp.dot(nxt, Wz[...], precision=PREC) + jnp.dot(enc, Uz[...], precision=PREC) + bz[...])
        r = jax.nn.sigmoid(jnp.dot(nxt, Wr[...], precision=PREC) + jnp.dot(enc, Ur[...], precision=PREC) + br[...])
        c = jnp.tanh(jnp.dot(nxt, Wh[...], precision=PREC) + jnp.dot(r * enc, Uh[...], precision=PREC) + bh[...])
        enc = z * enc + (1.0 - z) * c

    adjm = adj_ref[0]
    dh = jnp.tanh(jnp.dot(enc, decW1[...], precision=PREC) + decb1[...])
    pred_row = lax.dot_general(decW2[...], dh, (((0,), (1,)), ((), ())), precision=PREC) + decb2[...]
    w = adjm * pred_row - BIG * (1.0 - adjm)

    zmax = jnp.max(w, axis=1, keepdims=True)
    lo = zmax - 1.0
    hi = zmax
    for _ in range(BISECT_ITERS):
        mid = 0.5 * (lo + hi)
        fsum = jnp.sum(jnp.maximum(w - mid, 0.0), axis=1, keepdims=True)
        gt = fsum > 1.0
        lo = jnp.where(gt, mid, lo)
        hi = jnp.where(gt, hi, mid)
    sup = (w > lo).astype(jnp.float32)
    kk = jnp.sum(sup, axis=1, keepdims=True)
    ss = jnp.sum(w * sup, axis=1, keepdims=True)
    tau = (ss - 1.0) / kk
    fw = jnp.maximum(w - tau, 0.0) * adjm

    dem_row = dem_ref[0]
    r_row = jnp.maximum(dem_row, 0.0)
    for _ in range(FLOW_ITERS - 1):
        r_col = jnp.transpose(r_row)
        inflow = jnp.sum(fw * r_col, axis=0, keepdims=True)
        r_row = jnp.maximum(inflow + dem_row, 0.0)
    r_col = jnp.transpose(r_row)
    flow_cost = jnp.sum((fw * fw) * (r_col * r_col))

    du = jnp.tanh(jnp.dot(enc, duW1[...], precision=PREC) + dub1[...])
    dv_col = jnp.dot(du, duW2[...], precision=PREC) + dub2[...]
    dv_row = lax.dot_general(duW2[...], du, (((0,), (1,)), ((), ())), precision=PREC) + dub2[...]
    dd = adjm * (dv_col - dv_row)
    dd01 = DUAL_STEP * dd
    fl = jnp.zeros((N, N), jnp.float32)
    vel = jnp.zeros((N, N), jnp.float32)
    for _ in range(DUAL_ITERS):
        vel = DUAL_MOM * vel + (dd01 - (2.0 * DUAL_STEP) * fl)
        fl = jnp.maximum(fl + vel, 0.0)
    dual_demand = jnp.sum(dv_row * dem_row)
    dual_cost = jnp.sum(fl * fl - dd * fl) - dual_demand
    out_ref[...] = jnp.broadcast_to(flow_cost - dual_cost, (1, 1, 128))


def kernel(node_features, node_embeddings, demands, adj, neighborhoods, params):
    p = params
    x = jnp.concatenate([node_embeddings, node_features], axis=-1)
    dem_row = jnp.transpose(demands, (0, 2, 1))
    Wv = p['Wv'].reshape(K * H, D_ENC, DH)
    a_src = p['a_src'].reshape(K * H, DH)
    a_dst = p['a_dst'].reshape(K * H, DH)

    def b2(b):
        return b.reshape(1, -1)

    args = [x, dem_row, adj, neighborhoods,
            p['enc_W1'], b2(p['enc_b1']), p['enc_W2'], b2(p['enc_b2']),
            Wv, a_src, a_dst, p['W_comb'], b2(p['b_comb']),
            p['Wz'], p['Uz'], b2(p['bz']), p['Wr'], p['Ur'], b2(p['br']),
            p['Wh'], p['Uh'], b2(p['bh']),
            p['dec_W1'], b2(p['dec_b1']), p['dec_W2'], b2(p['dec_b2']),
            p['dual_W1'], b2(p['dual_b1']), p['dual_W2'], b2(p['dual_b2'])]

    n_batch_args = 4
    in_specs = []
    for idx, a in enumerate(args):
        if idx < n_batch_args:
            blk = (1,) + a.shape[1:]
            in_specs.append(
                pl.BlockSpec(blk, lambda b, _nd=a.ndim: (b,) + (0,) * (_nd - 1)))
        else:
            in_specs.append(
                pl.BlockSpec(a.shape, lambda b, _nd=a.ndim: (0,) * _nd))

    out = pl.pallas_call(
        _fwd_kernel,
        grid=(B,),
        in_specs=in_specs,
        out_specs=pl.BlockSpec((1, 1, 128), lambda b: (b, 0, 0)),
        out_shape=jax.ShapeDtypeStruct((B, 1, 128), jnp.float32),
        compiler_params=pltpu.CompilerParams(dimension_semantics=("parallel",)),
    )(*args)
    return out[:, 0, 0]
